# baseline (speedup 1.0000x reference)
import functools

import jax
import jax.numpy as jnp
from jax import lax
from jax.experimental import pallas as pl
from jax.experimental.pallas import tpu as pltpu
from jax.experimental.pallas import tpu_sc as plsc

NC = 2
NS = 16
NW = NC * NS
CHUNK = 128


def _sc_gather(u3, i3, user_table, item_table, n_chunks, emb):
    mesh = plsc.VectorSubcoreMesh(core_axis_name="c", subcore_axis_name="s")
    out_t = (
        jax.ShapeDtypeStruct((NW, n_chunks, CHUNK, emb), jnp.float32),
        jax.ShapeDtypeStruct((NW, n_chunks, CHUNK, emb), jnp.float32),
    )

    @functools.partial(
        pl.kernel,
        mesh=mesh,
        out_type=out_t,
        scratch_types=[
            pltpu.VMEM((n_chunks, CHUNK), jnp.int32),
            pltpu.VMEM((n_chunks, CHUNK), jnp.int32),
            pltpu.VMEM((n_chunks, CHUNK, emb), jnp.float32),
            pltpu.VMEM((n_chunks, CHUNK, emb), jnp.float32),
            pltpu.SemaphoreType.DMA,
        ],
        compiler_params=pltpu.CompilerParams(use_tc_tiling_on_sc=False),
    )
    def k(u_hbm, i_hbm, ut_hbm, it_hbm, ue_out, ie_out,
          uidx_v, iidx_v, urows_v, irows_v, sem):
        wid = lax.axis_index("s") * NC + lax.axis_index("c")
        pltpu.sync_copy(u_hbm.at[wid], uidx_v)
        pltpu.sync_copy(i_hbm.at[wid], iidx_v)
        copies = []
        for j in range(n_chunks):
            copies.append(pltpu.async_copy(ut_hbm.at[uidx_v.at[j]], urows_v.at[j], sem))
            copies.append(pltpu.async_copy(it_hbm.at[iidx_v.at[j]], irows_v.at[j], sem))
        for c in copies:
            c.wait()
        pltpu.sync_copy(urows_v, ue_out.at[wid])
        pltpu.sync_copy(irows_v, ie_out.at[wid])

    return k(u3, i3, user_table, item_table)


def _mlp_body(ue_ref, ie_ref, w1a_ref, w1b_ref, b1_ref, w2t_ref, b2_ref,
              w3_ref, b3_ref, out_ref):
    h1 = (jnp.dot(ue_ref[...], w1a_ref[...], preferred_element_type=jnp.float32)
          + jnp.dot(ie_ref[...], w1b_ref[...], preferred_element_type=jnp.float32)
          + b1_ref[...])
    h1 = jnp.maximum(h1, 0.0)
    h2 = jnp.dot(h1, w2t_ref[...], preferred_element_type=jnp.float32) + b2_ref[...]
    h2 = jnp.maximum(h2, 0.0)
    logit = jnp.sum(h2 * w3_ref[...], axis=1, keepdims=True) + b3_ref[0, 0]
    out_ref[...] = jax.nn.sigmoid(logit)


def _mlp(ue, ie, W1, b1, W2, b2, W3, b3, interpret=False):
    B, emb = ue.shape
    w1t = W1.T
    w1a, w1b = w1t[:emb], w1t[emb:]
    w2t = W2.T
    b1r = b1.reshape(1, -1)
    b2r = b2.reshape(1, -1)
    w3r = W3.reshape(1, -1)
    b3r = b3.reshape(1, 1)

    bb = 2048
    grid = (B // bb,)
    h1 = W1.shape[0]
    h2 = W2.shape[0]
    fixed = lambda shape: pl.BlockSpec(shape, lambda j: (0, 0))
    out = pl.pallas_call(
        _mlp_body,
        grid=grid,
        in_specs=[
            pl.BlockSpec((bb, emb), lambda j: (j, 0)),
            pl.BlockSpec((bb, emb), lambda j: (j, 0)),
            fixed((emb, h1)),
            fixed((emb, h1)),
            fixed((1, h1)),
            fixed((h1, h2)),
            fixed((1, h2)),
            fixed((1, h2)),
            fixed((1, 1)),
        ],
        out_specs=pl.BlockSpec((bb, 1), lambda j: (j, 0)),
        out_shape=jax.ShapeDtypeStruct((B, 1), jnp.float32),
        interpret=interpret,
    )(ue, ie, w1a, w1b, b1r, w2t, b2r, w3r, b3r)
    return jnp.squeeze(out, axis=-1)


def kernel(u, i, user_table, item_table, W1, b1, W2, b2, W3, b3):
    B = u.shape[0]
    emb = user_table.shape[1]
    n_chunks = B // (NW * CHUNK)
    u3 = u.astype(jnp.int32).reshape(NW, n_chunks, CHUNK)
    i3 = i.astype(jnp.int32).reshape(NW, n_chunks, CHUNK)
    ue4, ie4 = _sc_gather(u3, i3, user_table, item_table, n_chunks, emb)
    ue = ue4.reshape(B, emb)
    ie = ie4.reshape(B, emb)
    return _mlp(ue, ie, W1, b1, W2, b2, W3, b3)
